# baseline (speedup 1.0000x reference)
import functools

import jax
import jax.numpy as jnp
from jax import lax
from jax.experimental import pallas as pl
from jax.experimental.pallas import tpu as pltpu
from jax.experimental.pallas import tpu_sc as plsc

BATCH = 16384
EMB = 16
NROWS = 1000000

_NC, _NS = 2, 16
_NW = _NC * _NS
_RANGE = 31232
_CHUNK = 2048
_NFULL = _RANGE // _CHUNK
_REM = _RANGE - _NFULL * _CHUNK
_EXTRA = 512
_TAILN = NROWS % 128
_TAIL_START = NROWS - _TAILN
_WL_CAP = 768
_OB_ROWS = 128
_SC_BATCH = 16
_DUMMY = BATCH
OUT_ROWS = BATCH + _SC_BATCH


def _scan_range(idx_v, start, end, wlc_v, wlp_v):
    lanes = lax.iota(jnp.int32, 16)

    def block(b, ptr):
        i0 = idx_v[pl.ds(b * 32, 16)]
        i1 = idx_v[pl.ds(b * 32 + 16, 16)]
        m0 = jnp.logical_and(i0 >= start, i0 < end)
        m1 = jnp.logical_and(i1 >= start, i1 < end)
        cs0 = plsc.cumsum(m0.astype(jnp.int32))
        cs1 = plsc.cumsum(m1.astype(jnp.int32))
        p0 = ptr + cs0 - 1
        n0 = ptr + cs0[15]
        p1 = n0 + cs1 - 1
        plsc.store_scatter(wlc_v, [p0], i0 - start, mask=m0)
        plsc.store_scatter(wlp_v, [p0], b * 32 + lanes, mask=m0)
        plsc.store_scatter(wlc_v, [p1], i1 - start, mask=m1)
        plsc.store_scatter(wlp_v, [p1], b * 32 + 16 + lanes, mask=m1)
        return n0 + cs1[15]

    return lax.fori_loop(0, BATCH // 32, block, jnp.int32(0))


def _drain(nb, obuf_v, rid_v, out_hbm, sem_o):
    def drain(kb, carry):
        del carry
        pltpu.make_async_copy(obuf_v, out_hbm.at[rid_v], sem_o).wait()
        return 0

    lax.fori_loop(0, nb, drain, 0)


def _process_chunk(chunk_v, csize, cbase, cnt, wlc_v, wlp_v, w2c_v, w2p_v,
                   obuf_v, rid_v, out_hbm, sem_o, nb_prev):
    lanes = lax.iota(jnp.int32, 16)

    def filt(b, ptr):
        cols = wlc_v[pl.ds(b * 16, 16)]
        poss = wlp_v[pl.ds(b * 16, 16)]
        valid = b * 16 + lanes < cnt
        m = jnp.logical_and(
            valid, jnp.logical_and(cols >= cbase, cols < cbase + csize))
        cs = plsc.cumsum(m.astype(jnp.int32))
        pos = ptr + cs - 1
        plsc.store_scatter(w2c_v, [pos], cols - cbase, mask=m)
        plsc.store_scatter(w2p_v, [pos], poss, mask=m)
        return ptr + cs[15]

    cnt2 = lax.fori_loop(0, (cnt + 15) // 16, filt, jnp.int32(0))
    nb = (cnt2 + _SC_BATCH - 1) // _SC_BATCH

    _drain(nb_prev, obuf_v, rid_v, out_hbm, sem_o)

    dummy = jnp.full((16,), _DUMMY, jnp.int32)
    for k in range(_OB_ROWS // 16):
        rid_v[pl.ds(k * 16, 16)] = dummy

    def extract(kb, carry):
        del carry
        base = kb * 16
        valid = base + lanes < cnt2
        cols = jnp.where(valid, w2c_v[pl.ds(base, 16)], 0)
        poss = jnp.where(valid, w2p_v[pl.ds(base, 16)], _DUMMY)
        rid_v[pl.ds(base, 16)] = poss
        rows = base + lanes
        for c in range(EMB):
            cvec = jnp.full((16,), c, jnp.int32)
            vals = plsc.load_gather(chunk_v, [cvec, cols])
            plsc.store_scatter(obuf_v, [rows, cvec], vals)
        return 0

    lax.fori_loop(0, nb, extract, 0)
    pltpu.make_async_copy(obuf_v, out_hbm.at[rid_v], sem_o).start()
    return jnp.int32(1)


def _gather_one_table(idx_v, tabt_hbm, tail_hbm, out_hbm, start, extra,
                      wlc_v, wlp_v, w2c_v, w2p_v, chunk_v, tailv_v, obuf_v,
                      rid_v, sem_a, sem_b, sem_o0, sem_o1):
    cnt = _scan_range(idx_v, start, start + _RANGE + extra, wlc_v, wlp_v)

    def cp_in(buf, off, width, sem):
        return pltpu.make_async_copy(
            tabt_hbm.at[:, pl.ds(start + off, width)],
            buf.at[:, pl.ds(0, width)], sem)

    def body(ci, carry):
        ne, no = carry

        def run(buf, sem, semo, nprev):
            cp_in(chunk_v.at[buf], ci * _CHUNK, _CHUNK, sem).wait()
            nb = _process_chunk(chunk_v.at[buf], _CHUNK, ci * _CHUNK, cnt,
                                wlc_v, wlp_v, w2c_v, w2p_v, obuf_v.at[buf],
                                rid_v.at[buf], out_hbm, semo, nprev)

            @pl.when(ci + 2 < _NFULL)
            def _():
                cp_in(chunk_v.at[buf], (ci + 2) * _CHUNK, _CHUNK, sem).start()

            return nb

        even = ci % 2 == 0
        ne2 = lax.cond(even, lambda: run(0, sem_a, sem_o0, ne), lambda: ne)
        no2 = lax.cond(even, lambda: no, lambda: run(1, sem_b, sem_o1, no))
        return ne2, no2

    cp_in(chunk_v.at[0], 0, _CHUNK, sem_a).start()
    cp_in(chunk_v.at[1], _CHUNK, _CHUNK, sem_b).start()
    zero = jnp.int32(0)
    ne, no = lax.fori_loop(0, _NFULL, body, (zero, zero))

    rem = cp_in(chunk_v.at[0], _NFULL * _CHUNK, _REM, sem_a)
    rem.start()
    rem.wait()
    ne = _process_chunk(chunk_v.at[0], _REM, _NFULL * _CHUNK, cnt,
                        wlc_v, wlp_v, w2c_v, w2p_v, obuf_v.at[0], rid_v.at[0],
                        out_hbm, sem_o0, ne)

    def w31():
        ex = cp_in(chunk_v.at[0], _RANGE, _EXTRA, sem_a)
        ex.start()
        ex.wait()
        n1 = _process_chunk(chunk_v.at[0], _EXTRA, _RANGE, cnt,
                            wlc_v, wlp_v, w2c_v, w2p_v, obuf_v.at[0],
                            rid_v.at[0], out_hbm, sem_o0, ne)
        pltpu.sync_copy(tail_hbm, tailv_v)
        return _process_chunk(tailv_v, _TAILN, _RANGE + _EXTRA, cnt,
                              wlc_v, wlp_v, w2c_v, w2p_v, obuf_v.at[0],
                              rid_v.at[0], out_hbm, sem_o0, n1)

    ne = lax.cond(extra > 0, w31, lambda: ne)

    _drain(ne, obuf_v.at[0], rid_v.at[0], out_hbm, sem_o0)
    _drain(no, obuf_v.at[1], rid_v.at[1], out_hbm, sem_o1)


def _sc_gather(user_hbm, item_hbm, ut_hbm, it_hbm, utail_hbm, itail_hbm,
               ue_hbm, ie_hbm,
               idx_v, wlc_v, wlp_v, w2c_v, w2p_v, chunk_v, tailv_v,
               obuf_v, rid_v, sem_a, sem_b, sem_o0, sem_o1):
    wid = lax.axis_index("s") * _NC + lax.axis_index("c")
    start = wid * _RANGE
    extra = jnp.where(wid == _NW - 1, _EXTRA + _TAILN, 0)
    pltpu.sync_copy(user_hbm, idx_v)
    _gather_one_table(idx_v, ut_hbm, utail_hbm, ue_hbm, start, extra,
                      wlc_v, wlp_v, w2c_v, w2p_v, chunk_v, tailv_v, obuf_v,
                      rid_v, sem_a, sem_b, sem_o0, sem_o1)
    pltpu.sync_copy(item_hbm, idx_v)
    _gather_one_table(idx_v, it_hbm, itail_hbm, ie_hbm, start, extra,
                      wlc_v, wlp_v, w2c_v, w2p_v, chunk_v, tailv_v, obuf_v,
                      rid_v, sem_a, sem_b, sem_o0, sem_o1)


@functools.cache
def _gather_call():
    return pl.kernel(
        _sc_gather,
        mesh=plsc.VectorSubcoreMesh(core_axis_name="c", subcore_axis_name="s"),
        compiler_params=pltpu.CompilerParams(needs_layout_passes=False),
        out_type=[
            jax.ShapeDtypeStruct((OUT_ROWS, 128), jnp.float32),
            jax.ShapeDtypeStruct((OUT_ROWS, 128), jnp.float32),
        ],
        scratch_types=[
            pltpu.VMEM((BATCH,), jnp.int32),
            pltpu.VMEM((_WL_CAP,), jnp.int32),
            pltpu.VMEM((_WL_CAP,), jnp.int32),
            pltpu.VMEM((_OB_ROWS,), jnp.int32),
            pltpu.VMEM((_OB_ROWS,), jnp.int32),
            pltpu.VMEM((2, EMB, _CHUNK), jnp.float32),
            pltpu.VMEM((EMB, _TAILN), jnp.float32),
            pltpu.VMEM((2, _OB_ROWS, 128), jnp.float32),
            pltpu.VMEM((2, _OB_ROWS), jnp.int32),
            pltpu.SemaphoreType.DMA,
            pltpu.SemaphoreType.DMA,
            pltpu.SemaphoreType.DMA,
            pltpu.SemaphoreType.DMA,
        ],
    )


def _mlp_body(ue_ref, ie_ref, w1u_ref, w1i_ref, b1_ref, w2_ref, b2_ref,
              wo_ref, bo_ref, out_ref):
    h = (
        jnp.dot(ue_ref[:, :EMB], w1u_ref[...], preferred_element_type=jnp.float32)
        + jnp.dot(ie_ref[:, :EMB], w1i_ref[...], preferred_element_type=jnp.float32)
        + b1_ref[...]
    )
    h = jnp.maximum(h, 0.0)
    h = jnp.dot(h, w2_ref[...], preferred_element_type=jnp.float32) + b2_ref[...]
    h = jnp.maximum(h, 0.0)
    logits = jnp.dot(h, wo_ref[...], preferred_element_type=jnp.float32) + bo_ref[...]
    out_ref[...] = jax.nn.sigmoid(logits)


def kernel(user, item, user_table, item_table, W1, b1, W2, b2, Wo, bo):
    ue_r, ie_r = _gather_call()(
        user, item, user_table.T, item_table.T,
        user_table[_TAIL_START:, :].T, item_table[_TAIL_START:, :].T)
    out = pl.pallas_call(
        _mlp_body,
        grid=(1,),
        in_specs=[
            pl.BlockSpec((BATCH, 128), lambda i: (0, 0)),
            pl.BlockSpec((BATCH, 128), lambda i: (0, 0)),
            pl.BlockSpec((EMB, EMB), lambda i: (0, 0)),
            pl.BlockSpec((EMB, EMB), lambda i: (0, 0)),
            pl.BlockSpec((1, EMB), lambda i: (0, 0)),
            pl.BlockSpec((EMB, 8), lambda i: (0, 0)),
            pl.BlockSpec((1, 8), lambda i: (0, 0)),
            pl.BlockSpec((8, 1), lambda i: (0, 0)),
            pl.BlockSpec((1, 1), lambda i: (0, 0)),
        ],
        out_specs=pl.BlockSpec((BATCH, 1), lambda i: (0, 0)),
        out_shape=jax.ShapeDtypeStruct((BATCH, 1), jnp.float32),
    )(
        ue_r,
        ie_r,
        W1[:, :EMB].T,
        W1[:, EMB:].T,
        b1.reshape(1, -1),
        W2.T,
        b2.reshape(1, -1),
        Wo.T,
        bo.reshape(1, 1),
    )
    return out.reshape(BATCH)


# speedup vs baseline: 22.7200x; 22.7200x over previous
import functools

import jax
import jax.numpy as jnp
from jax import lax
from jax.experimental import pallas as pl
from jax.experimental.pallas import tpu as pltpu
from jax.experimental.pallas import tpu_sc as plsc

BATCH = 16384
EMB = 16
NROWS = 1000000

_NC, _NS = 2, 16
_NW = _NC * _NS
_RANGE = 31232
_CHUNK = 2048
_NFULL = _RANGE // _CHUNK
_REM = _RANGE - _NFULL * _CHUNK
_EXTRA = 512
_TAILN = NROWS % 128
_TAIL_START = NROWS - _TAILN
_WL_CAP = 768
_OB_ROWS = 128
_SC_BATCH = 16
_DUMMY = BATCH
OUT_ROWS = BATCH + _SC_BATCH


def _scan_range(idx_v, start, end, wlc_v, wlp_v):
    lanes = lax.iota(jnp.int32, 16)

    def block(b, ptr):
        i0 = idx_v[pl.ds(b * 32, 16)]
        i1 = idx_v[pl.ds(b * 32 + 16, 16)]
        m0 = jnp.logical_and(i0 >= start, i0 < end)
        m1 = jnp.logical_and(i1 >= start, i1 < end)
        cs0 = plsc.cumsum(m0.astype(jnp.int32))
        cs1 = plsc.cumsum(m1.astype(jnp.int32))
        p0 = ptr + cs0 - 1
        n0 = ptr + cs0[15]
        p1 = n0 + cs1 - 1
        plsc.store_scatter(wlc_v, [p0], i0 - start, mask=m0)
        plsc.store_scatter(wlp_v, [p0], b * 32 + lanes, mask=m0)
        plsc.store_scatter(wlc_v, [p1], i1 - start, mask=m1)
        plsc.store_scatter(wlp_v, [p1], b * 32 + 16 + lanes, mask=m1)
        return n0 + cs1[15]

    return lax.fori_loop(0, BATCH // 32, block, jnp.int32(0))


def _drain(nb, obuf_v, rid_v, out_hbm, sem_o):
    def drain(kb, carry):
        del carry
        pltpu.make_async_copy(
            obuf_v.at[pl.ds(kb * _SC_BATCH, _SC_BATCH)],
            out_hbm.at[rid_v.at[kb]],
            sem_o,
        ).wait()
        return 0

    lax.fori_loop(0, nb, drain, 0)


def _process_chunk(chunk_v, csize, cbase, cnt, wlc_v, wlp_v, w2c_v, w2p_v,
                   obuf_v, rid_v, out_hbm, sem_o, nb_prev):
    lanes = lax.iota(jnp.int32, 16)

    def filt(b, ptr):
        cols = wlc_v[pl.ds(b * 16, 16)]
        poss = wlp_v[pl.ds(b * 16, 16)]
        valid = b * 16 + lanes < cnt
        m = jnp.logical_and(
            valid, jnp.logical_and(cols >= cbase, cols < cbase + csize))
        cs = plsc.cumsum(m.astype(jnp.int32))
        pos = ptr + cs - 1
        plsc.store_scatter(w2c_v, [pos], cols - cbase, mask=m)
        plsc.store_scatter(w2p_v, [pos], poss, mask=m)
        return ptr + cs[15]

    cnt2 = lax.fori_loop(0, (cnt + 15) // 16, filt, jnp.int32(0))
    nb = (cnt2 + _SC_BATCH - 1) // _SC_BATCH

    _drain(nb_prev, obuf_v, rid_v, out_hbm, sem_o)

    def extract(kb, carry):
        del carry
        base = kb * 16
        valid = base + lanes < cnt2
        cols = jnp.where(valid, w2c_v[pl.ds(base, 16)], 0)
        poss = jnp.where(valid, w2p_v[pl.ds(base, 16)], _DUMMY + lanes)
        rid_v[kb, :] = poss
        rows = base + lanes
        for c in range(EMB):
            cvec = jnp.full((16,), c, jnp.int32)
            vals = plsc.load_gather(chunk_v, [cvec, cols])
            plsc.store_scatter(obuf_v, [rows, cvec], vals)
        return 0

    lax.fori_loop(0, nb, extract, 0)

    def fire(kb, carry):
        del carry
        pltpu.make_async_copy(
            obuf_v.at[pl.ds(kb * _SC_BATCH, _SC_BATCH)],
            out_hbm.at[rid_v.at[kb]],
            sem_o,
        ).start()
        return 0

    lax.fori_loop(0, nb, fire, 0)
    return nb


def _gather_one_table(idx_v, tabt_hbm, tail_hbm, out_hbm, start, extra,
                      wlc_v, wlp_v, w2c_v, w2p_v, chunk_v, tailv_v, obuf_v,
                      rid_v, sem_a, sem_b, sem_o0, sem_o1):
    cnt = _scan_range(idx_v, start, start + _RANGE + extra, wlc_v, wlp_v)

    def cp_in(buf, off, width, sem):
        return pltpu.make_async_copy(
            tabt_hbm.at[:, pl.ds(start + off, width)],
            buf.at[:, pl.ds(0, width)], sem)

    def body(ci, carry):
        ne, no = carry

        def run(buf, sem, semo, nprev):
            cp_in(chunk_v.at[buf], ci * _CHUNK, _CHUNK, sem).wait()
            nb = _process_chunk(chunk_v.at[buf], _CHUNK, ci * _CHUNK, cnt,
                                wlc_v, wlp_v, w2c_v, w2p_v, obuf_v.at[buf],
                                rid_v.at[buf], out_hbm, semo, nprev)

            @pl.when(ci + 2 < _NFULL)
            def _():
                cp_in(chunk_v.at[buf], (ci + 2) * _CHUNK, _CHUNK, sem).start()

            return nb

        even = ci % 2 == 0
        ne2 = lax.cond(even, lambda: run(0, sem_a, sem_o0, ne), lambda: ne)
        no2 = lax.cond(even, lambda: no, lambda: run(1, sem_b, sem_o1, no))
        return ne2, no2

    cp_in(chunk_v.at[0], 0, _CHUNK, sem_a).start()
    cp_in(chunk_v.at[1], _CHUNK, _CHUNK, sem_b).start()
    zero = jnp.int32(0)
    ne, no = lax.fori_loop(0, _NFULL, body, (zero, zero))

    rem = cp_in(chunk_v.at[0], _NFULL * _CHUNK, _REM, sem_a)
    rem.start()
    rem.wait()
    ne = _process_chunk(chunk_v.at[0], _REM, _NFULL * _CHUNK, cnt,
                        wlc_v, wlp_v, w2c_v, w2p_v, obuf_v.at[0], rid_v.at[0],
                        out_hbm, sem_o0, ne)

    def w31():
        ex = cp_in(chunk_v.at[0], _RANGE, _EXTRA, sem_a)
        ex.start()
        ex.wait()
        n1 = _process_chunk(chunk_v.at[0], _EXTRA, _RANGE, cnt,
                            wlc_v, wlp_v, w2c_v, w2p_v, obuf_v.at[0],
                            rid_v.at[0], out_hbm, sem_o0, ne)
        pltpu.sync_copy(tail_hbm, tailv_v)
        return _process_chunk(tailv_v, _TAILN, _RANGE + _EXTRA, cnt,
                              wlc_v, wlp_v, w2c_v, w2p_v, obuf_v.at[0],
                              rid_v.at[0], out_hbm, sem_o0, n1)

    ne = lax.cond(extra > 0, w31, lambda: ne)

    _drain(ne, obuf_v.at[0], rid_v.at[0], out_hbm, sem_o0)
    _drain(no, obuf_v.at[1], rid_v.at[1], out_hbm, sem_o1)


def _sc_gather(user_hbm, item_hbm, ut_hbm, it_hbm, utail_hbm, itail_hbm,
               ue_hbm, ie_hbm,
               idx_v, wlc_v, wlp_v, w2c_v, w2p_v, chunk_v, tailv_v,
               obuf_v, rid_v, sem_a, sem_b, sem_o0, sem_o1):
    wid = lax.axis_index("s") * _NC + lax.axis_index("c")
    start = wid * _RANGE
    extra = jnp.where(wid == _NW - 1, _EXTRA + _TAILN, 0)
    pltpu.sync_copy(user_hbm, idx_v)
    _gather_one_table(idx_v, ut_hbm, utail_hbm, ue_hbm, start, extra,
                      wlc_v, wlp_v, w2c_v, w2p_v, chunk_v, tailv_v, obuf_v,
                      rid_v, sem_a, sem_b, sem_o0, sem_o1)
    pltpu.sync_copy(item_hbm, idx_v)
    _gather_one_table(idx_v, it_hbm, itail_hbm, ie_hbm, start, extra,
                      wlc_v, wlp_v, w2c_v, w2p_v, chunk_v, tailv_v, obuf_v,
                      rid_v, sem_a, sem_b, sem_o0, sem_o1)


@functools.cache
def _gather_call():
    return pl.kernel(
        _sc_gather,
        mesh=plsc.VectorSubcoreMesh(core_axis_name="c", subcore_axis_name="s"),
        compiler_params=pltpu.CompilerParams(needs_layout_passes=False),
        out_type=[
            jax.ShapeDtypeStruct((OUT_ROWS, 128), jnp.float32),
            jax.ShapeDtypeStruct((OUT_ROWS, 128), jnp.float32),
        ],
        scratch_types=[
            pltpu.VMEM((BATCH,), jnp.int32),
            pltpu.VMEM((_WL_CAP,), jnp.int32),
            pltpu.VMEM((_WL_CAP,), jnp.int32),
            pltpu.VMEM((_OB_ROWS,), jnp.int32),
            pltpu.VMEM((_OB_ROWS,), jnp.int32),
            pltpu.VMEM((2, EMB, _CHUNK), jnp.float32),
            pltpu.VMEM((EMB, _TAILN), jnp.float32),
            pltpu.VMEM((2, _OB_ROWS, 128), jnp.float32),
            pltpu.VMEM((2, _OB_ROWS // _SC_BATCH, _SC_BATCH), jnp.int32),
            pltpu.SemaphoreType.DMA,
            pltpu.SemaphoreType.DMA,
            pltpu.SemaphoreType.DMA,
            pltpu.SemaphoreType.DMA,
        ],
    )


def _mlp_body(ue_ref, ie_ref, w1u_ref, w1i_ref, b1_ref, w2_ref, b2_ref,
              wo_ref, bo_ref, out_ref):
    h = (
        jnp.dot(ue_ref[:, :EMB], w1u_ref[...], preferred_element_type=jnp.float32)
        + jnp.dot(ie_ref[:, :EMB], w1i_ref[...], preferred_element_type=jnp.float32)
        + b1_ref[...]
    )
    h = jnp.maximum(h, 0.0)
    h = jnp.dot(h, w2_ref[...], preferred_element_type=jnp.float32) + b2_ref[...]
    h = jnp.maximum(h, 0.0)
    logits = jnp.dot(h, wo_ref[...], preferred_element_type=jnp.float32) + bo_ref[...]
    out_ref[...] = jax.nn.sigmoid(logits)


def kernel(user, item, user_table, item_table, W1, b1, W2, b2, Wo, bo):
    ue_r, ie_r = _gather_call()(
        user, item, user_table.T, item_table.T,
        user_table[_TAIL_START:, :].T, item_table[_TAIL_START:, :].T)
    out = pl.pallas_call(
        _mlp_body,
        grid=(1,),
        in_specs=[
            pl.BlockSpec((BATCH, 128), lambda i: (0, 0)),
            pl.BlockSpec((BATCH, 128), lambda i: (0, 0)),
            pl.BlockSpec((EMB, EMB), lambda i: (0, 0)),
            pl.BlockSpec((EMB, EMB), lambda i: (0, 0)),
            pl.BlockSpec((1, EMB), lambda i: (0, 0)),
            pl.BlockSpec((EMB, 8), lambda i: (0, 0)),
            pl.BlockSpec((1, 8), lambda i: (0, 0)),
            pl.BlockSpec((8, 1), lambda i: (0, 0)),
            pl.BlockSpec((1, 1), lambda i: (0, 0)),
        ],
        out_specs=pl.BlockSpec((BATCH, 1), lambda i: (0, 0)),
        out_shape=jax.ShapeDtypeStruct((BATCH, 1), jnp.float32),
    )(
        ue_r,
        ie_r,
        W1[:, :EMB].T,
        W1[:, EMB:].T,
        b1.reshape(1, -1),
        W2.T,
        b2.reshape(1, -1),
        Wo.T,
        bo.reshape(1, 1),
    )
    return out.reshape(BATCH)
